# baseline (speedup 1.0000x reference)
import functools

import jax
import jax.numpy as jnp
from jax.experimental import pallas as pl
from jax.experimental.pallas import tpu as pltpu

_T_BLK = 512
_NCHUNK = 64


def _prep(cb_ref, cbm2_ref, c2_ref, cbt_ref):
    cb = cb_ref[...]
    cbm2_ref[...] = cb * -2.0
    c2 = jnp.sum(cb * cb, axis=1, keepdims=True)
    c2_ref[...] = jnp.broadcast_to(c2, c2_ref.shape)
    K, D = cb.shape
    kc = K // _NCHUNK
    cb3 = cb.reshape(_NCHUNK, kc, D)
    cbt_ref[...] = jnp.transpose(cb3, (0, 2, 1)).reshape(_NCHUNK * D, kc)


def _vq_block(in_ref, cbt_ref, cbm2_ref, c2_ref, out_st_ref, out_q_ref, *, K):
    xb = in_ref[0]
    tb = xb.shape[1]
    x2 = jnp.sum(xb * xb, axis=0, keepdims=True)
    kc = K // _NCHUNK
    ramp = jax.lax.broadcasted_iota(jnp.int32, (kc, tb), 0).astype(jnp.float32)
    cmins = []
    cidxs = []
    for c in range(_NCHUNK):
        rows = pl.ds(c * kc, kc)
        e2 = jax.lax.dot_general(cbm2_ref[rows, :], xb,
                                 (((1,), (0,)), ((), ())),
                                 preferred_element_type=jnp.float32)
        d = (x2 + e2) + c2_ref[rows, :]
        m = jnp.min(d, axis=0, keepdims=True)
        hit = jnp.where(d == m, ramp, float(kc))
        cmins.append(m)
        cidxs.append(jnp.min(hit, axis=0, keepdims=True) + float(c * kc))
    mind = functools.reduce(jnp.minimum, cmins)
    idx = functools.reduce(
        jnp.minimum,
        [jnp.where(m == mind, ci, float(K)) for m, ci in zip(cmins, cidxs)])
    g = jnp.floor(idx * (1.0 / kc))
    r = idx - g * kc
    onehot_r = jnp.where(ramp == r, 1.0, 0.0)
    D = out_q_ref.shape[1]
    tb = out_q_ref.shape[2]
    y = jax.lax.dot_general(cbt_ref[...], onehot_r, (((1,), (0,)), ((), ())),
                            preferred_element_type=jnp.float32)
    giota = jax.lax.broadcasted_iota(
        jnp.int32, (_NCHUNK, 1, tb), 0).astype(jnp.float32)
    og = jnp.where(giota == g[None, :, :], 1.0, 0.0)
    q = jnp.sum(y.reshape(_NCHUNK, D, tb) * og, axis=0)
    out_st_ref[0] = xb + (q - xb)
    out_q_ref[0] = q


@jax.jit
def kernel(input, codebook):
    B, D, T = input.shape
    K, _ = codebook.shape
    kc = K // _NCHUNK
    cbm2, c2b, cbt = pl.pallas_call(
        _prep,
        out_shape=[
            jax.ShapeDtypeStruct((K, D), jnp.float32),
            jax.ShapeDtypeStruct((K, _T_BLK), jnp.float32),
            jax.ShapeDtypeStruct((_NCHUNK * D, kc), jnp.float32),
        ],
    )(codebook)
    out_shape = jax.ShapeDtypeStruct((B, D, T), jnp.float32)
    fn = pl.pallas_call(
        functools.partial(_vq_block, K=K),
        grid=(B, T // _T_BLK),
        in_specs=[
            pl.BlockSpec((1, D, _T_BLK), lambda b, t: (b, 0, t)),
            pl.BlockSpec((_NCHUNK * D, kc), lambda b, t: (0, 0)),
            pl.BlockSpec((K, D), lambda b, t: (0, 0)),
            pl.BlockSpec((K, _T_BLK), lambda b, t: (0, 0)),
        ],
        out_specs=[
            pl.BlockSpec((1, D, _T_BLK), lambda b, t: (b, 0, t)),
            pl.BlockSpec((1, D, _T_BLK), lambda b, t: (b, 0, t)),
        ],
        out_shape=[out_shape, out_shape],
        compiler_params=pltpu.CompilerParams(
            dimension_semantics=("parallel", "parallel")),
    )
    qst, q = fn(input, cbt, cbm2, c2b)
    return (qst, q)


# speedup vs baseline: 1.1029x; 1.1029x over previous
import functools

import jax
import jax.numpy as jnp
from jax.experimental import pallas as pl
from jax.experimental.pallas import tpu as pltpu

_T_BLK = 512
_NCHUNK = 16


def _prep(cb_ref, cbm2_ref, c2_ref, cbt_ref):
    cb = cb_ref[...]
    cbm2_ref[...] = cb * -2.0
    c2 = jnp.sum(cb * cb, axis=1, keepdims=True)
    c2_ref[...] = jnp.broadcast_to(c2, c2_ref.shape)
    K, D = cb.shape
    kc = K // _NCHUNK
    cb3 = cb.reshape(_NCHUNK, kc, D)
    cbt_ref[...] = jnp.transpose(cb3, (0, 2, 1)).reshape(_NCHUNK * D, kc)


def _vq_block(in_ref, cbt_ref, cbm2_ref, c2_ref, out_st_ref, out_q_ref, *, K):
    xb = in_ref[0]
    tb = xb.shape[1]
    x2 = jnp.sum(xb * xb, axis=0, keepdims=True)
    kc = K // _NCHUNK
    ramp = jax.lax.broadcasted_iota(jnp.int32, (kc, tb), 0).astype(jnp.float32)
    cmins = []
    cidxs = []
    for c in range(_NCHUNK):
        rows = pl.ds(c * kc, kc)
        e2 = jax.lax.dot_general(cbm2_ref[rows, :], xb,
                                 (((1,), (0,)), ((), ())),
                                 preferred_element_type=jnp.float32)
        d = (x2 + e2) + c2_ref[rows, :]
        m = jnp.min(d, axis=0, keepdims=True)
        hit = jnp.where(d == m, ramp, float(kc))
        cmins.append(m)
        cidxs.append(jnp.min(hit, axis=0, keepdims=True) + float(c * kc))
    mind = functools.reduce(jnp.minimum, cmins)
    idx = functools.reduce(
        jnp.minimum,
        [jnp.where(m == mind, ci, float(K)) for m, ci in zip(cmins, cidxs)])
    g = jnp.floor(idx * (1.0 / kc))
    r = idx - g * kc
    onehot_r = jnp.where(ramp == r, 1.0, 0.0)
    D = out_q_ref.shape[1]
    tb = out_q_ref.shape[2]
    y = jax.lax.dot_general(cbt_ref[...], onehot_r, (((1,), (0,)), ((), ())),
                            preferred_element_type=jnp.float32)
    giota = jax.lax.broadcasted_iota(
        jnp.int32, (_NCHUNK, 1, tb), 0).astype(jnp.float32)
    og = jnp.where(giota == g[None, :, :], 1.0, 0.0)
    q = jnp.sum(y.reshape(_NCHUNK, D, tb) * og, axis=0)
    out_st_ref[0] = xb + (q - xb)
    out_q_ref[0] = q


@jax.jit
def kernel(input, codebook):
    B, D, T = input.shape
    K, _ = codebook.shape
    kc = K // _NCHUNK
    cbm2, c2b, cbt = pl.pallas_call(
        _prep,
        out_shape=[
            jax.ShapeDtypeStruct((K, D), jnp.float32),
            jax.ShapeDtypeStruct((K, _T_BLK), jnp.float32),
            jax.ShapeDtypeStruct((_NCHUNK * D, kc), jnp.float32),
        ],
    )(codebook)
    out_shape = jax.ShapeDtypeStruct((B, D, T), jnp.float32)
    fn = pl.pallas_call(
        functools.partial(_vq_block, K=K),
        grid=(B, T // _T_BLK),
        in_specs=[
            pl.BlockSpec((1, D, _T_BLK), lambda b, t: (b, 0, t)),
            pl.BlockSpec((_NCHUNK * D, kc), lambda b, t: (0, 0)),
            pl.BlockSpec((K, D), lambda b, t: (0, 0)),
            pl.BlockSpec((K, _T_BLK), lambda b, t: (0, 0)),
        ],
        out_specs=[
            pl.BlockSpec((1, D, _T_BLK), lambda b, t: (b, 0, t)),
            pl.BlockSpec((1, D, _T_BLK), lambda b, t: (b, 0, t)),
        ],
        out_shape=[out_shape, out_shape],
        compiler_params=pltpu.CompilerParams(
            dimension_semantics=("parallel", "parallel")),
    )
    qst, q = fn(input, cbt, cbm2, c2b)
    return (qst, q)


# speedup vs baseline: 1.1228x; 1.0181x over previous
import functools

import jax
import jax.numpy as jnp
from jax.experimental import pallas as pl
from jax.experimental.pallas import tpu as pltpu

_T_BLK = 512
_NCHUNK = 16


def _prep(cb_ref, cbm2_ref, c2_ref, cbt_ref):
    cb = cb_ref[...]
    cbm2_ref[...] = cb * -2.0
    c2 = jnp.sum(cb * cb, axis=1, keepdims=True)
    c2_ref[...] = jnp.broadcast_to(c2, c2_ref.shape)
    K, D = cb.shape
    kc = K // _NCHUNK
    cb3 = cb.reshape(_NCHUNK, kc, D)
    cbt_ref[...] = jnp.transpose(cb3, (0, 2, 1)).reshape(_NCHUNK * D, kc)


def _vq_block(in_ref, cbt_ref, cbm2_ref, c2_ref, out_st_ref, out_q_ref, *, K):
    xb = in_ref[0]
    tb = xb.shape[1]
    x2 = jnp.sum(xb * xb, axis=0, keepdims=True)
    kc = K // _NCHUNK
    ramp = jax.lax.broadcasted_iota(jnp.int32, (kc, tb), 0).astype(jnp.float32)
    cmins = []
    cidxs = []
    for c in range(_NCHUNK):
        rows = pl.ds(c * kc, kc)
        e2 = jax.lax.dot_general(cbm2_ref[rows, :], xb,
                                 (((1,), (0,)), ((), ())),
                                 preferred_element_type=jnp.float32)
        c2c = jnp.broadcast_to(c2_ref[rows, 0:1], (kc, tb))
        d = (x2 + e2) + c2c
        m = jnp.min(d, axis=0, keepdims=True)
        hit = jnp.where(d == m, ramp, float(kc))
        cmins.append(m)
        cidxs.append(jnp.min(hit, axis=0, keepdims=True) + float(c * kc))
    mind = functools.reduce(jnp.minimum, cmins)
    idx = functools.reduce(
        jnp.minimum,
        [jnp.where(m == mind, ci, float(K)) for m, ci in zip(cmins, cidxs)])
    g = jnp.floor(idx * (1.0 / kc))
    r = idx - g * kc
    onehot_r = jnp.where(ramp == r, 1.0, 0.0)
    D = out_q_ref.shape[1]
    tb = out_q_ref.shape[2]
    y = jax.lax.dot_general(cbt_ref[...], onehot_r, (((1,), (0,)), ((), ())),
                            preferred_element_type=jnp.float32)
    giota = jax.lax.broadcasted_iota(
        jnp.int32, (_NCHUNK, 1, tb), 0).astype(jnp.float32)
    og = jnp.where(giota == g[None, :, :], 1.0, 0.0)
    q = jnp.sum(y.reshape(_NCHUNK, D, tb) * og, axis=0)
    out_st_ref[0] = xb + (q - xb)
    out_q_ref[0] = q


@jax.jit
def kernel(input, codebook):
    B, D, T = input.shape
    K, _ = codebook.shape
    kc = K // _NCHUNK
    cbm2, c2b, cbt = pl.pallas_call(
        _prep,
        out_shape=[
            jax.ShapeDtypeStruct((K, D), jnp.float32),
            jax.ShapeDtypeStruct((K, 128), jnp.float32),
            jax.ShapeDtypeStruct((_NCHUNK * D, kc), jnp.float32),
        ],
    )(codebook)
    out_shape = jax.ShapeDtypeStruct((B, D, T), jnp.float32)
    fn = pl.pallas_call(
        functools.partial(_vq_block, K=K),
        grid=(B, T // _T_BLK),
        in_specs=[
            pl.BlockSpec((1, D, _T_BLK), lambda b, t: (b, 0, t)),
            pl.BlockSpec((_NCHUNK * D, kc), lambda b, t: (0, 0)),
            pl.BlockSpec((K, D), lambda b, t: (0, 0)),
            pl.BlockSpec((K, 128), lambda b, t: (0, 0)),
        ],
        out_specs=[
            pl.BlockSpec((1, D, _T_BLK), lambda b, t: (b, 0, t)),
            pl.BlockSpec((1, D, _T_BLK), lambda b, t: (b, 0, t)),
        ],
        out_shape=[out_shape, out_shape],
        compiler_params=pltpu.CompilerParams(
            dimension_semantics=("parallel", "parallel")),
    )
    qst, q = fn(input, cbt, cbm2, c2b)
    return (qst, q)
